# baseline (speedup 1.0000x reference)
import jax
import jax.numpy as jnp
from jax.experimental import pallas as pl


def _body(x_ref, w_ref, b_ref, o_ref):
    x = x_ref[...]
    w = w_ref[0]
    y = jax.lax.dot_general(
        x, w, (((1,), (1,)), ((), ())), preferred_element_type=jnp.float32
    )
    o_ref[...] = y + b_ref[0]


def kernel(inputs, W, b):
    N, E, D = inputs.shape
    BN = 512
    x2 = inputs.reshape(N, E * D)
    b3 = b.reshape(E, 1, D)

    out = pl.pallas_call(
        _body,
        grid=(E, N // BN),
        in_specs=[
            pl.BlockSpec((BN, D), lambda e, i: (i, e)),
            pl.BlockSpec((1, D, D), lambda e, i: (e, 0, 0)),
            pl.BlockSpec((1, 1, D), lambda e, i: (e, 0, 0)),
        ],
        out_specs=pl.BlockSpec((BN, D), lambda e, i: (i, e)),
        out_shape=jax.ShapeDtypeStruct((N, E * D), jnp.float32),
    )(x2, W, b3)
    return out.reshape(N, E, D)


# speedup vs baseline: 1.9148x; 1.9148x over previous
import functools

import jax
import jax.numpy as jnp
from jax.experimental import pallas as pl


def _body(x_ref, w_ref, b_ref, o_ref, *, E, D):
    x = x_ref[...].astype(jnp.bfloat16)
    for e in range(E):
        xe = x[:, e * D:(e + 1) * D]
        we = w_ref[e]
        y = jax.lax.dot_general(
            xe, we, (((1,), (1,)), ((), ())),
            preferred_element_type=jnp.float32,
        )
        o_ref[:, e * D:(e + 1) * D] = y + b_ref[e]


def kernel(inputs, W, b):
    N, E, D = inputs.shape
    BN = 256
    x2 = inputs.reshape(N, E * D)
    w_bf = W.astype(jnp.bfloat16)

    out = pl.pallas_call(
        functools.partial(_body, E=E, D=D),
        grid=(N // BN,),
        in_specs=[
            pl.BlockSpec((BN, E * D), lambda i: (i, 0)),
            pl.BlockSpec((E, D, D), lambda i: (0, 0, 0)),
            pl.BlockSpec((E, D), lambda i: (0, 0)),
        ],
        out_specs=pl.BlockSpec((BN, E * D), lambda i: (i, 0)),
        out_shape=jax.ShapeDtypeStruct((N, E * D), jnp.float32),
    )(x2, w_bf, b)
    return out.reshape(N, E, D)


# speedup vs baseline: 3.2807x; 1.7134x over previous
import functools

import jax
import jax.numpy as jnp
from jax.experimental import pallas as pl


def _body(x_ref, w_ref, b_ref, o_ref, *, E):
    x = x_ref[...].astype(jnp.bfloat16)
    for e in range(E):
        xe = x[:, e, :]
        we = w_ref[e]
        y = jax.lax.dot_general(
            xe, we, (((1,), (1,)), ((), ())),
            preferred_element_type=jnp.float32,
        )
        o_ref[:, e, :] = y + b_ref[e]


def kernel(inputs, W, b):
    N, E, D = inputs.shape
    BN = 256
    w_bf = W.astype(jnp.bfloat16)

    return pl.pallas_call(
        functools.partial(_body, E=E),
        grid=(N // BN,),
        in_specs=[
            pl.BlockSpec((BN, E, D), lambda i: (i, 0, 0)),
            pl.BlockSpec((E, D, D), lambda i: (0, 0, 0)),
            pl.BlockSpec((E, D), lambda i: (0, 0)),
        ],
        out_specs=pl.BlockSpec((BN, E, D), lambda i: (i, 0, 0)),
        out_shape=jax.ShapeDtypeStruct((N, E, D), jnp.float32),
    )(inputs, w_bf, b)


# speedup vs baseline: 3.2931x; 1.0038x over previous
import functools

import jax
import jax.numpy as jnp
from jax.experimental import pallas as pl


def _body(x_ref, w_ref, b_ref, o_ref, *, E):
    x = x_ref[...].astype(jnp.bfloat16)
    for e in range(E):
        xe = x[:, e, :]
        we = w_ref[e]
        y = jax.lax.dot_general(
            xe, we, (((1,), (1,)), ((), ())),
            preferred_element_type=jnp.float32,
        )
        o_ref[:, e, :] = y + b_ref[e]


def kernel(inputs, W, b):
    N, E, D = inputs.shape
    BN = 128
    w_bf = W.astype(jnp.bfloat16)

    return pl.pallas_call(
        functools.partial(_body, E=E),
        grid=(N // BN,),
        in_specs=[
            pl.BlockSpec((BN, E, D), lambda i: (i, 0, 0)),
            pl.BlockSpec((E, D, D), lambda i: (0, 0, 0)),
            pl.BlockSpec((E, D), lambda i: (0, 0)),
        ],
        out_specs=pl.BlockSpec((BN, E, D), lambda i: (i, 0, 0)),
        out_shape=jax.ShapeDtypeStruct((N, E, D), jnp.float32),
    )(inputs, w_bf, b)
